# baseline (speedup 1.0000x reference)
import functools

import jax
import jax.numpy as jnp
import numpy as np
from jax import lax
from jax.experimental import pallas as pl
from jax.experimental.pallas import tpu as pltpu
from jax.experimental.pallas import tpu_sc as plsc

D = 256
NF = 19
N_ACTIVE = 12288
N_SIDE = 6144
N_TOTAL = N_ACTIVE + N_SIDE
KPAD = 128

_B = 512
_GRID = N_TOTAL // _B

_NC, _NS, _L = 2, 16, 16
_NW = _NC * _NS
_C = 64
_NCHUNK = N_TOTAL // _C
_PER_W = _NCHUNK // _NW
_K1 = (N_ACTIVE // _C) // _NW

_EQ_SEGS = ((3, 3, 18), (4, 7, 21), (5, 2, 28), (6, 2, 30), (7, 2, 32),
            (8, 8, 34), (9, 4, 42), (10, 2, 46), (11, 16, 48))
_SEL = np.zeros((NF, KPAD), np.float32)
_TGT = np.full((1, KPAD), -1.0, np.float32)
for _f, _n, _base in _EQ_SEGS:
    for _t in range(_n):
        _SEL[_f, _base + _t] = 1.0
        _TGT[0, _base + _t] = float(_t)


def _tc_body(feats_ref, wcat_ref, bias_ref, sel_ref, tgt_ref, out_ref):
    feats = feats_ref[...]
    hp = feats[:, 0:1].astype(jnp.float32)
    maxhp = jnp.maximum(feats[:, 1:2], 1).astype(jnp.float32)
    ratio = jnp.clip(hp / maxhp, 0.0, 1.0)
    token = (1023.0 * ratio).astype(jnp.int32)
    lvl = feats[:, 2:3]
    m0 = feats[:, 15:16]
    m1 = feats[:, 16:17]
    m2 = feats[:, 17:18]

    c = lax.broadcasted_iota(jnp.int32, (_B, KPAD), 1)
    bitsrc = jnp.where(c < 10, token, lvl)
    sh = jnp.clip(jnp.where(c < 10, c, c - 11), 0, 31)
    bits = (lax.shift_right_logical(bitsrc, sh) & 1).astype(jnp.float32)
    fsel = jnp.dot(feats.astype(jnp.float32), sel_ref[...],
                   preferred_element_type=jnp.float32)
    eq = (jnp.abs(fsel - tgt_ref[...]) < 0.5).astype(jnp.float32)
    cm = c - 64
    mv = (((m0 == cm) | (m1 == cm) | (m2 == cm)) & (c < 68)).astype(jnp.float32)
    code = jnp.where(c == 10, ratio,
                     jnp.where(c < 18, bits,
                               jnp.where(c < 64, eq, mv)))
    out_ref[...] = jnp.dot(code, wcat_ref[...],
                           preferred_element_type=jnp.float32) + bias_ref[...]


def _tc_call(feats, wcat, bias):
    return pl.pallas_call(
        _tc_body,
        grid=(_GRID,),
        in_specs=[
            pl.BlockSpec((_B, NF), lambda i: (i, 0)),
            pl.BlockSpec((KPAD, D), lambda i: (0, 0)),
            pl.BlockSpec((1, D), lambda i: (0, 0)),
            pl.BlockSpec((NF, KPAD), lambda i: (0, 0)),
            pl.BlockSpec((1, KPAD), lambda i: (0, 0)),
        ],
        out_specs=pl.BlockSpec((_B, D), lambda i: (i, 0)),
        out_shape=jax.ShapeDtypeStruct((N_TOTAL, D), jnp.float32),
    )(feats, wcat, bias, jnp.asarray(_SEL), jnp.asarray(_TGT))


@functools.cache
def _sc_gather():
    mesh = plsc.VectorSubcoreMesh(core_axis_name="c", subcore_axis_name="s",
                                  num_cores=_NC)

    @functools.partial(
        pl.kernel,
        mesh=mesh,
        out_type=(
            jax.ShapeDtypeStruct((N_ACTIVE, D), jnp.float32),
            jax.ShapeDtypeStruct((N_SIDE, D), jnp.float32),
        ),
        scratch_types=[
            pltpu.VMEM((_C,), jnp.int32),
            pltpu.VMEM((_C,), jnp.int32),
            pltpu.VMEM((_C,), jnp.int32),
            pltpu.VMEM((_C, D), jnp.float32),
            pltpu.VMEM((_C, D), jnp.float32),
            pltpu.VMEM((_C, D), jnp.float32),
            pltpu.VMEM((_C, D), jnp.float32),
            pltpu.SemaphoreType.DMA,
        ],
    )
    def sc_fn(table_hbm, idx_hbm, tc_hbm, out1_hbm, out2_hbm,
              i0, i1, i2, r0, r1, r2, acc, sem):
        w = lax.axis_index("s") * _NC + lax.axis_index("c")
        for k in range(_PER_W):
            base = (k * _NW + w) * _C
            pltpu.sync_copy(idx_hbm.at[0, pl.ds(base, _C)], i0)
            pltpu.sync_copy(idx_hbm.at[1, pl.ds(base, _C)], i1)
            pltpu.sync_copy(idx_hbm.at[2, pl.ds(base, _C)], i2)
            pltpu.sync_copy(tc_hbm.at[pl.ds(base, _C)], acc)
            cp0 = pltpu.async_copy(table_hbm.at[i0], r0, sem)
            cp1 = pltpu.async_copy(table_hbm.at[i1], r1, sem)
            cp2 = pltpu.async_copy(table_hbm.at[i2], r2, sem)
            cp0.wait()
            cp1.wait()
            cp2.wait()

            def add_rows(i, carry):
                for j in range(D // _L):
                    sl = pl.ds(j * _L, _L)
                    acc[i, sl] = acc[i, sl] + r0[i, sl] + r1[i, sl] + r2[i, sl]
                return carry

            lax.fori_loop(0, _C, add_rows, 0)
            if k < _K1:
                pltpu.sync_copy(acc, out1_hbm.at[pl.ds(base, _C)])
            else:
                pltpu.sync_copy(acc, out2_hbm.at[pl.ds(base - N_ACTIVE, _C)])

    return sc_fn


def kernel(active_entities, side_entities, W_onehot, b_onehot, W_species,
           b_species, W_ability, b_ability, W_item, b_item, W_moveset,
           b_moveset):
    feats = jnp.concatenate(
        [active_entities.reshape(N_ACTIVE, NF), side_entities], axis=0)
    wcat = jnp.concatenate(
        [W_onehot, W_moveset, jnp.zeros((KPAD - 68, D), jnp.float32)], axis=0)
    bias = (b_onehot + b_species + b_ability + b_item + b_moveset).reshape(1, D)
    table = jnp.concatenate(
        [W_species, W_ability, jnp.zeros((1, D), jnp.float32), W_item], axis=0)
    abil = feats[:, 13]
    idx = jnp.stack([
        feats[:, 12],
        jnp.where(abil < 320, abil + 1280, 1600),
        feats[:, 14] + 1601,
    ]).astype(jnp.int32)
    tc_out = _tc_call(feats, wcat, bias)
    out1, out2 = _sc_gather()(table, idx, tc_out)
    return out1.reshape(1024, 12, D), out2
